# baseline (speedup 1.0000x reference)
import functools

import jax
import jax.numpy as jnp
from jax import lax
from jax.experimental import pallas as pl
from jax.experimental.pallas import tpu as pltpu
from jax.experimental.pallas import tpu_sc as plsc

N_NODES = 10000
N_EDGES = 320000
NC = 2
NS = 16
NW = NC * NS
LANES = 128
CH = 80
EPT_PAD = CH * LANES
NPAD = 10112
RPT = NPAD // NS
BCH = 24
CHMAX = 120
CH_SLOW = 109
CH_FAST = 48
SLOW_CORE = 0
NBLK = CHMAX // BCH

_mesh = plsc.VectorSubcoreMesh(core_axis_name="c", subcore_axis_name="s")


@functools.partial(
    pl.kernel,
    out_type=jax.ShapeDtypeStruct((NC, NPAD, 128), jnp.float32),
    mesh=_mesh,
    scratch_types=[
        pltpu.VMEM((CH, LANES), jnp.int32),
        pltpu.VMEM((LANES, 128), jnp.float32),
        pltpu.VMEM_SHARED((NPAD, 128), jnp.float32),
        pltpu.SemaphoreType.DMA,
        pltpu.SemaphoreType.DMA,
    ],
)
def _sc_degree(dst_hbm, ones_hbm, zeros_hbm, degp_hbm, dst_v, ones_v,
               acc_sh, sd0, sd1):
    sd = [sd0, sd1]
    c = lax.axis_index("c")
    s = lax.axis_index("s")
    pltpu.sync_copy(dst_hbm.at[c].at[s], dst_v)
    pltpu.sync_copy(ones_hbm, ones_v)
    row0 = s * RPT
    pltpu.sync_copy(zeros_hbm.at[pl.ds(row0, RPT)],
                    acc_sh.at[pl.ds(row0, RPT)])
    plsc.subcore_barrier()

    def wait_sd(b):
        pltpu.make_async_copy(ones_hbm, ones_v, sd[b]).wait()

    for j in range(CH):
        pltpu.async_copy(ones_v, acc_sh.at[dst_v.at[j]], sd[j % 2],
                         add=True)
        if j >= 1:
            wait_sd((j - 1) % 2)
    wait_sd((CH - 1) % 2)
    plsc.subcore_barrier()
    pltpu.sync_copy(acc_sh.at[pl.ds(row0, RPT)],
                    degp_hbm.at[c].at[pl.ds(row0, RPT)])


@functools.partial(
    pl.kernel,
    out_type=jax.ShapeDtypeStruct((NC, NPAD, 128), jnp.float32),
    mesh=_mesh,
    scratch_types=[
        pltpu.VMEM((BCH, LANES), jnp.int32),
        pltpu.VMEM((BCH, LANES), jnp.int32),
        pltpu.VMEM((2, LANES, 128), jnp.float32),
        pltpu.VMEM_SHARED((NPAD, 128), jnp.float32),
        pltpu.SemaphoreType.DMA,
        pltpu.SemaphoreType.DMA,
    ],
)
def _sc_aggregate(src_hbm, dst_hbm, y_hbm, zeros_hbm, zp_hbm,
                  src_v, dst_v, rows_v, acc_sh, sg0, sg1):
    sg = [sg0, sg1]
    c = lax.axis_index("c")
    s = lax.axis_index("s")
    ch_c = jnp.where(c == SLOW_CORE, CH_SLOW, CH_FAST)
    sblocks = src_hbm.at[c].at[s]
    dblocks = dst_hbm.at[c].at[s]

    def wait_rows(sem):
        pltpu.make_async_copy(y_hbm.at[pl.ds(0, LANES)], rows_v.at[0],
                              sem).wait()

    pltpu.sync_copy(sblocks.at[pl.ds(0, BCH)], src_v)
    pltpu.sync_copy(dblocks.at[pl.ds(0, BCH)], dst_v)
    pltpu.async_copy(y_hbm.at[src_v.at[0]], rows_v.at[0], sg[0])
    row0 = s * RPT
    pltpu.sync_copy(zeros_hbm.at[pl.ds(row0, RPT)],
                    acc_sh.at[pl.ds(row0, RPT)])
    plsc.subcore_barrier()

    for j in range(CHMAX):
        k, jj = divmod(j, BCH)
        b = j % 2

        @pl.when(j < ch_c)
        def _():
            if jj == 0 and k > 0:
                pltpu.sync_copy(sblocks.at[pl.ds(k * BCH, BCH)], src_v)
                pltpu.sync_copy(dblocks.at[pl.ds(k * BCH, BCH)], dst_v)
                pltpu.async_copy(y_hbm.at[src_v.at[0]], rows_v.at[b], sg[b])
            wait_rows(sg[b])
            if jj + 1 < BCH:
                @pl.when(j + 1 < ch_c)
                def _():
                    pltpu.async_copy(y_hbm.at[src_v.at[jj + 1]],
                                     rows_v.at[1 - b], sg[1 - b])
            pltpu.sync_copy(rows_v.at[b], acc_sh.at[dst_v.at[jj]], add=True)
    plsc.subcore_barrier()
    pltpu.sync_copy(acc_sh.at[pl.ds(row0, RPT)],
                    zp_hbm.at[c].at[pl.ds(row0, RPT)])


def _tc_prep_body(degp_ref, x_ref, dinv_ref, y1_ref):
    deg = degp_ref[0] + degp_ref[1] + 1.0
    dinv = lax.rsqrt(deg)
    dinv_ref[...] = dinv
    y1_ref[...] = x_ref[...] * dinv[:N_NODES]


def _tc_mid_body(zp_ref, y1_ref, dinv_ref, w1_ref, b1_ref, y2_ref):
    d = dinv_ref[:N_NODES]
    ax = d * (zp_ref[0, :N_NODES, :] + zp_ref[1, :N_NODES, :] + y1_ref[...])
    h = jnp.maximum(
        jnp.dot(ax, w1_ref[...], preferred_element_type=jnp.float32)
        + b1_ref[...], 0.0)
    y2_ref[...] = h * d


def _tc_head_body(zp_ref, y2_ref, dinv_ref, wmu_ref, bmu_ref, wls_ref,
                  bls_ref, mu_ref, ls_ref):
    d = dinv_ref[:N_NODES]
    ah = d * (zp_ref[0, :N_NODES, :] + zp_ref[1, :N_NODES, :] + y2_ref[...])
    mu_ref[...] = (
        jnp.dot(ah, wmu_ref[...], preferred_element_type=jnp.float32)
        + bmu_ref[...])
    ls_ref[...] = (
        jnp.dot(ah, wls_ref[...], preferred_element_type=jnp.float32)
        + bls_ref[...])


def kernel(x, edge_index, W1, b1, Wmu, bmu, Wls, bls):
    src = edge_index[0].astype(jnp.int32)
    dst = edge_index[1].astype(jnp.int32)
    pad = NW * EPT_PAD - N_EDGES
    srcf = jnp.concatenate([src, jnp.zeros((pad,), jnp.int32)])
    dstf = jnp.concatenate([dst, jnp.full((pad,), N_NODES, jnp.int32)])
    srcp = srcf.reshape(NC, NS, CH, LANES)
    dstp = dstf.reshape(NC, NS, CH, LANES)

    nslow = NS * CH_SLOW * LANES
    ntot = NS * (CH_SLOW + CH_FAST) * LANES

    def _chpad(n):
        return ((0, 0), (0, CHMAX - n), (0, 0))

    def _split(e, fill):
        ef = jnp.concatenate([e, jnp.full((ntot - N_EDGES,), fill,
                                          jnp.int32)])
        es = jnp.pad(ef[:nslow].reshape(NS, CH_SLOW, LANES),
                     _chpad(CH_SLOW), constant_values=fill)
        eb = jnp.pad(ef[nslow:].reshape(NS, CH_FAST, LANES),
                     _chpad(CH_FAST), constant_values=fill)
        halves = [es, eb] if SLOW_CORE == 0 else [eb, es]
        return jnp.stack(halves)

    srca = _split(src, 0)
    dsta = _split(dst, N_NODES)

    ones128 = jnp.ones((LANES, 128), jnp.float32)
    zeros128 = jnp.zeros((NPAD, 128), jnp.float32)

    degp = _sc_degree(dstp, ones128, zeros128)

    dinv, y1 = pl.pallas_call(
        _tc_prep_body,
        out_shape=[
            jax.ShapeDtypeStruct((NPAD, 128), jnp.float32),
            jax.ShapeDtypeStruct((N_NODES, 128), jnp.float32),
        ],
    )(degp, x)

    zp1 = _sc_aggregate(srca, dsta, y1, zeros128)

    y2 = pl.pallas_call(
        _tc_mid_body,
        out_shape=jax.ShapeDtypeStruct((N_NODES, 128), jnp.float32),
    )(zp1, y1, dinv, W1, b1.reshape(1, 128))

    zp2 = _sc_aggregate(srca, dsta, y2, zeros128)

    mu, logstd = pl.pallas_call(
        _tc_head_body,
        out_shape=[
            jax.ShapeDtypeStruct((N_NODES, 64), jnp.float32),
            jax.ShapeDtypeStruct((N_NODES, 64), jnp.float32),
        ],
    )(zp2, y2, dinv, Wmu, bmu.reshape(1, 64), Wls, bls.reshape(1, 64))

    return (mu, logstd)


# speedup vs baseline: 1.0642x; 1.0642x over previous
import functools

import jax
import jax.numpy as jnp
from jax import lax
from jax.experimental import pallas as pl
from jax.experimental.pallas import tpu as pltpu
from jax.experimental.pallas import tpu_sc as plsc

N_NODES = 10000
N_EDGES = 320000
NC = 2
NS = 16
NW = NC * NS
LANES = 128
CH = 80
EPT_PAD = CH * LANES
NPAD = 10112
RPT = NPAD // NS
BCH = 24
CHMAX = 96
CH_SLOW = 96
CH_FAST = 61
SLOW_CORE = 0
NBLK = CHMAX // BCH

_mesh = plsc.VectorSubcoreMesh(core_axis_name="c", subcore_axis_name="s")


@functools.partial(
    pl.kernel,
    out_type=jax.ShapeDtypeStruct((NC, NPAD, 128), jnp.float32),
    mesh=_mesh,
    scratch_types=[
        pltpu.VMEM((CH, LANES), jnp.int32),
        pltpu.VMEM((LANES, 128), jnp.float32),
        pltpu.VMEM_SHARED((NPAD, 128), jnp.float32),
        pltpu.SemaphoreType.DMA,
        pltpu.SemaphoreType.DMA,
    ],
)
def _sc_degree(dst_hbm, ones_hbm, zeros_hbm, degp_hbm, dst_v, ones_v,
               acc_sh, sd0, sd1):
    sd = [sd0, sd1]
    c = lax.axis_index("c")
    s = lax.axis_index("s")
    pltpu.sync_copy(dst_hbm.at[c].at[s], dst_v)
    pltpu.sync_copy(ones_hbm, ones_v)
    row0 = s * RPT
    pltpu.sync_copy(zeros_hbm.at[pl.ds(row0, RPT)],
                    acc_sh.at[pl.ds(row0, RPT)])
    plsc.subcore_barrier()

    def wait_sd(b):
        pltpu.make_async_copy(ones_hbm, ones_v, sd[b]).wait()

    for j in range(CH):
        pltpu.async_copy(ones_v, acc_sh.at[dst_v.at[j]], sd[j % 2],
                         add=True)
        if j >= 1:
            wait_sd((j - 1) % 2)
    wait_sd((CH - 1) % 2)
    plsc.subcore_barrier()
    pltpu.sync_copy(acc_sh.at[pl.ds(row0, RPT)],
                    degp_hbm.at[c].at[pl.ds(row0, RPT)])


@functools.partial(
    pl.kernel,
    out_type=jax.ShapeDtypeStruct((NC, NPAD, 128), jnp.float32),
    mesh=_mesh,
    scratch_types=[
        pltpu.VMEM((BCH, LANES), jnp.int32),
        pltpu.VMEM((BCH, LANES), jnp.int32),
        pltpu.VMEM((2, LANES, 128), jnp.float32),
        pltpu.VMEM_SHARED((NPAD, 128), jnp.float32),
        pltpu.SemaphoreType.DMA,
        pltpu.SemaphoreType.DMA,
    ],
)
def _sc_aggregate(src_hbm, dst_hbm, y_hbm, zeros_hbm, zp_hbm,
                  src_v, dst_v, rows_v, acc_sh, sg0, sg1):
    sg = [sg0, sg1]
    c = lax.axis_index("c")
    s = lax.axis_index("s")
    ch_c = jnp.where(c == SLOW_CORE, CH_SLOW, CH_FAST)
    sblocks = src_hbm.at[c].at[s]
    dblocks = dst_hbm.at[c].at[s]

    def wait_rows(sem):
        pltpu.make_async_copy(y_hbm.at[pl.ds(0, LANES)], rows_v.at[0],
                              sem).wait()

    pltpu.sync_copy(sblocks.at[pl.ds(0, BCH)], src_v)
    pltpu.sync_copy(dblocks.at[pl.ds(0, BCH)], dst_v)
    pltpu.async_copy(y_hbm.at[src_v.at[0]], rows_v.at[0], sg[0])
    row0 = s * RPT
    pltpu.sync_copy(zeros_hbm.at[pl.ds(row0, RPT)],
                    acc_sh.at[pl.ds(row0, RPT)])
    plsc.subcore_barrier()

    for j in range(CHMAX):
        k, jj = divmod(j, BCH)
        b = j % 2

        @pl.when(j < ch_c)
        def _():
            if jj == 0 and k > 0:
                pltpu.sync_copy(sblocks.at[pl.ds(k * BCH, BCH)], src_v)
                pltpu.sync_copy(dblocks.at[pl.ds(k * BCH, BCH)], dst_v)
                pltpu.async_copy(y_hbm.at[src_v.at[0]], rows_v.at[b], sg[b])
            wait_rows(sg[b])
            if jj + 1 < BCH:
                @pl.when(j + 1 < ch_c)
                def _():
                    pltpu.async_copy(y_hbm.at[src_v.at[jj + 1]],
                                     rows_v.at[1 - b], sg[1 - b])
            pltpu.sync_copy(rows_v.at[b], acc_sh.at[dst_v.at[jj]], add=True)
    plsc.subcore_barrier()
    pltpu.sync_copy(acc_sh.at[pl.ds(row0, RPT)],
                    zp_hbm.at[c].at[pl.ds(row0, RPT)])


def _tc_prep_body(degp_ref, x_ref, dinv_ref, y1_ref):
    deg = degp_ref[0] + degp_ref[1] + 1.0
    dinv = lax.rsqrt(deg)
    dinv_ref[...] = dinv
    y1_ref[...] = x_ref[...] * dinv[:N_NODES]


def _tc_mid_body(zp_ref, y1_ref, dinv_ref, w1_ref, b1_ref, y2_ref):
    d = dinv_ref[:N_NODES]
    ax = d * (zp_ref[0, :N_NODES, :] + zp_ref[1, :N_NODES, :] + y1_ref[...])
    h = jnp.maximum(
        jnp.dot(ax, w1_ref[...], preferred_element_type=jnp.float32)
        + b1_ref[...], 0.0)
    y2_ref[...] = h * d


def _tc_head_body(zp_ref, y2_ref, dinv_ref, wmu_ref, bmu_ref, wls_ref,
                  bls_ref, mu_ref, ls_ref):
    d = dinv_ref[:N_NODES]
    ah = d * (zp_ref[0, :N_NODES, :] + zp_ref[1, :N_NODES, :] + y2_ref[...])
    mu_ref[...] = (
        jnp.dot(ah, wmu_ref[...], preferred_element_type=jnp.float32)
        + bmu_ref[...])
    ls_ref[...] = (
        jnp.dot(ah, wls_ref[...], preferred_element_type=jnp.float32)
        + bls_ref[...])


def kernel(x, edge_index, W1, b1, Wmu, bmu, Wls, bls):
    src = edge_index[0].astype(jnp.int32)
    dst = edge_index[1].astype(jnp.int32)
    pad = NW * EPT_PAD - N_EDGES
    srcf = jnp.concatenate([src, jnp.zeros((pad,), jnp.int32)])
    dstf = jnp.concatenate([dst, jnp.full((pad,), N_NODES, jnp.int32)])
    srcp = srcf.reshape(NC, NS, CH, LANES)
    dstp = dstf.reshape(NC, NS, CH, LANES)

    nslow = NS * CH_SLOW * LANES
    ntot = NS * (CH_SLOW + CH_FAST) * LANES

    def _chpad(n):
        return ((0, 0), (0, CHMAX - n), (0, 0))

    def _split(e, fill):
        ef = jnp.concatenate([e, jnp.full((ntot - N_EDGES,), fill,
                                          jnp.int32)])
        es = jnp.pad(ef[:nslow].reshape(NS, CH_SLOW, LANES),
                     _chpad(CH_SLOW), constant_values=fill)
        eb = jnp.pad(ef[nslow:].reshape(NS, CH_FAST, LANES),
                     _chpad(CH_FAST), constant_values=fill)
        halves = [es, eb] if SLOW_CORE == 0 else [eb, es]
        return jnp.stack(halves)

    srca = _split(src, 0)
    dsta = _split(dst, N_NODES)

    ones128 = jnp.ones((LANES, 128), jnp.float32)
    zeros128 = jnp.zeros((NPAD, 128), jnp.float32)

    degp = _sc_degree(dstp, ones128, zeros128)

    dinv, y1 = pl.pallas_call(
        _tc_prep_body,
        out_shape=[
            jax.ShapeDtypeStruct((NPAD, 128), jnp.float32),
            jax.ShapeDtypeStruct((N_NODES, 128), jnp.float32),
        ],
    )(degp, x)

    zp1 = _sc_aggregate(srca, dsta, y1, zeros128)

    y2 = pl.pallas_call(
        _tc_mid_body,
        out_shape=jax.ShapeDtypeStruct((N_NODES, 128), jnp.float32),
    )(zp1, y1, dinv, W1, b1.reshape(1, 128))

    zp2 = _sc_aggregate(srca, dsta, y2, zeros128)

    mu, logstd = pl.pallas_call(
        _tc_head_body,
        out_shape=[
            jax.ShapeDtypeStruct((N_NODES, 64), jnp.float32),
            jax.ShapeDtypeStruct((N_NODES, 64), jnp.float32),
        ],
    )(zp2, y2, dinv, Wmu, bmu.reshape(1, 64), Wls, bls.reshape(1, 64))

    return (mu, logstd)
